# Initial kernel scaffold; baseline (speedup 1.0000x reference)
import jax
import jax.numpy as jnp
from jax.experimental import pallas as pl


def kernel(surface_points, surface_normals, off_surface_points, near_surface_points, W1, b1, W2, b2):
    raise NotImplementedError("write your pallas kernel here")


# speedup vs baseline: 10.8736x; 10.8736x over previous
import jax
import jax.numpy as jnp
from jax.experimental import pallas as pl

_SDF_W = 7000.0
_EIK_W = 600.0
_ORI_W = 500.0
_NEAR_ORI_W = 10.0
_GRADN_W = 200.0

_N = 8192
_Q = 8192
_QB = 256
_SB = 128
_GRID = (2 * _Q) // _QB
_BIG = 3e38


def _loss_kernel(q_ref, spack_ref, npack_ref, srows_ref, nrows_ref,
                 w1p_ref, w1pt_ref, w2r_ref, b1_ref, b2_ref, total_ref):
    i = pl.program_id(0)

    @pl.when(i == 0)
    def _init():
        total_ref[:, :] = jnp.zeros((1, 1), jnp.float32)

    f32 = jnp.float32
    qblk = q_ref[:, :]
    qx = qblk[:, 0:1]
    qy = qblk[:, 1:2]
    qz = qblk[:, 2:3]
    q2 = qx * qx + qy * qy + qz * qz

    sx = spack_ref[0:1, :]
    sy = spack_ref[1:2, :]
    sz = spack_ref[2:3, :]
    s2 = spack_ref[3:4, :]

    d2 = (q2 + s2) - 2.0 * (qx * sx + qy * sy + qz * sz)
    m1 = jnp.min(d2, axis=1, keepdims=True)
    d2b = jnp.where(d2 <= m1, _BIG, d2)
    m2 = jnp.min(d2b, axis=1, keepdims=True)
    sel = (d2 <= m2).astype(f32)
    seln = jax.lax.dot_general(sel, npack_ref[:, :],
                               (((1,), (0,)), ((), ())),
                               preferred_element_type=f32)
    dotsum = (seln[:, 0:1] * qx + seln[:, 1:2] * qy + seln[:, 2:3] * qz
              - seln[:, 3:4])
    sgn = jnp.sign(dotsum)

    w1p = w1p_ref[:, :]
    w2r = w2r_ref[:, :]
    b1 = b1_ref[:, :]
    b2 = b2_ref[:, :]

    hq = jnp.tanh(jax.lax.dot_general(qblk, w1p, (((1,), (0,)), ((), ())),
                                      preferred_element_type=f32) + b1)
    predq = jnp.sum(hq * w2r, axis=1, keepdims=True) + b2
    ori = jnp.sum(jnp.maximum(-predq * sgn, 0.0))
    w_ori = jnp.where(i < (_Q // _QB), _ORI_W / _Q, _NEAR_ORI_W / _Q)

    sblk = srows_ref[:, :]
    h = jnp.tanh(jax.lax.dot_general(sblk, w1p, (((1,), (0,)), ((), ())),
                                     preferred_element_type=f32) + b1)
    preds = jnp.sum(h * w2r, axis=1, keepdims=True) + b2
    u = (1.0 - h * h) * w2r
    grad = jax.lax.dot_general(u, w1pt_ref[:, :], (((1,), (0,)), ((), ())),
                               preferred_element_type=f32)
    sdf_part = jnp.sum(preds * preds)
    g2 = jnp.sum(grad * grad, axis=1, keepdims=True)
    eik_part = jnp.sum((jnp.sqrt(g2) - 1.0) ** 2)
    gradn_part = jnp.sum((grad - nrows_ref[:, :]) ** 2)

    contrib = ((_SDF_W / _N) * sdf_part
               + (_EIK_W / _N) * eik_part
               + (_GRADN_W / (_N * 3)) * gradn_part
               + w_ori * ori)
    total_ref[:, :] += contrib.reshape(1, 1)


def kernel(surface_points, surface_normals, off_surface_points,
           near_surface_points, W1, b1, W2, b2):
    f32 = jnp.float32
    s = surface_points.astype(f32)
    n = surface_normals.astype(f32)
    s2 = jnp.sum(s * s, axis=1)
    snd = jnp.sum(s * n, axis=1)

    spack = jnp.concatenate(
        [s.T, s2[None, :], jnp.zeros((4, _N), f32)], axis=0)
    npack = jnp.concatenate(
        [n, snd[:, None], jnp.zeros((_N, 4), f32)], axis=1)
    srows = jnp.concatenate([s, jnp.zeros((_N, 5), f32)], axis=1)
    nrows = jnp.concatenate([n, jnp.zeros((_N, 5), f32)], axis=1)

    q = jnp.concatenate([off_surface_points, near_surface_points], axis=0)
    qrows = jnp.concatenate([q.astype(f32), jnp.zeros((2 * _Q, 5), f32)],
                            axis=1)

    H = W1.shape[1]
    w1p = jnp.concatenate([W1.astype(f32), jnp.zeros((5, H), f32)], axis=0)
    w1pt = w1p.T
    w2r = W2.astype(f32).reshape(1, H)
    b1r = b1.astype(f32).reshape(1, H)
    b2s = b2.astype(f32).reshape(1, 1)

    out_shape = jax.ShapeDtypeStruct((1, 1), f32)
    scalar_spec = pl.BlockSpec((1, 1), lambda i: (0, 0))
    total = pl.pallas_call(
        _loss_kernel,
        grid=(_GRID,),
        in_specs=[
            pl.BlockSpec((_QB, 8), lambda i: (i, 0)),
            pl.BlockSpec((8, _N), lambda i: (0, 0)),
            pl.BlockSpec((_N, 8), lambda i: (0, 0)),
            pl.BlockSpec((_SB, 8), lambda i: (i, 0)),
            pl.BlockSpec((_SB, 8), lambda i: (i, 0)),
            pl.BlockSpec((8, H), lambda i: (0, 0)),
            pl.BlockSpec((H, 8), lambda i: (0, 0)),
            pl.BlockSpec((1, H), lambda i: (0, 0)),
            pl.BlockSpec((1, H), lambda i: (0, 0)),
            scalar_spec,
        ],
        out_specs=scalar_spec,
        out_shape=out_shape,
    )(qrows, spack, npack, srows, nrows, w1p, w1pt, w2r, b1r, b2s)
    return total[0, 0]


# speedup vs baseline: 14.1239x; 1.2989x over previous
import jax
import jax.numpy as jnp
from jax.experimental import pallas as pl

_SDF_W = 7000.0
_EIK_W = 600.0
_ORI_W = 500.0
_NEAR_ORI_W = 10.0
_GRADN_W = 200.0

_N = 8192
_Q = 8192
_QB = 256
_SB = 128
_GRID = (2 * _Q) // _QB
_BIG = 3e38


def _loss_kernel(q_ref, spack_ref, npack_ref, srows_ref, nrows_ref,
                 w1p_ref, w1pt_ref, w2r_ref, b1_ref, b2_ref, total_ref):
    i = pl.program_id(0)

    @pl.when(i == 0)
    def _init():
        total_ref[:, :] = jnp.zeros((1, 1), jnp.float32)

    f32 = jnp.float32
    qblk = q_ref[:, :]
    qx = qblk[:, 0:1]
    qy = qblk[:, 1:2]
    qz = qblk[:, 2:3]

    d2 = jax.lax.dot_general(qblk, spack_ref[:, :], (((1,), (0,)), ((), ())),
                             preferred_element_type=f32)

    a1 = d2[:, 0:128]
    a2 = jnp.full((_QB, 128), _BIG, f32)
    for c in range(1, _N // 128):
        v = d2[:, 128 * c:128 * (c + 1)]
        hi = jnp.maximum(a1, v)
        a1 = jnp.minimum(a1, v)
        a2 = jnp.minimum(a2, hi)
    m1 = jnp.min(a1, axis=1, keepdims=True)
    m2a = jnp.min(jnp.where(a1 <= m1, _BIG, a1), axis=1, keepdims=True)
    m2b = jnp.min(a2, axis=1, keepdims=True)
    m2 = jnp.minimum(m2a, m2b)
    sel = (d2 <= m2).astype(f32)
    seln = jax.lax.dot_general(sel, npack_ref[:, :],
                               (((1,), (0,)), ((), ())),
                               preferred_element_type=f32)
    dotsum = (seln[:, 0:1] * qx + seln[:, 1:2] * qy + seln[:, 2:3] * qz
              - seln[:, 3:4])
    sgn = jnp.sign(dotsum)

    w1p = w1p_ref[:, :]
    w2r = w2r_ref[:, :]
    b1 = b1_ref[:, :]
    b2 = b2_ref[:, :]

    hq = jnp.tanh(jax.lax.dot_general(qblk, w1p, (((1,), (0,)), ((), ())),
                                      preferred_element_type=f32) + b1)
    predq = jnp.sum(hq * w2r, axis=1, keepdims=True) + b2
    ori = jnp.sum(jnp.maximum(-predq * sgn, 0.0))
    w_ori = jnp.where(i < (_Q // _QB), _ORI_W / _Q, _NEAR_ORI_W / _Q)

    sblk = srows_ref[:, :]
    h = jnp.tanh(jax.lax.dot_general(sblk, w1p, (((1,), (0,)), ((), ())),
                                     preferred_element_type=f32) + b1)
    preds = jnp.sum(h * w2r, axis=1, keepdims=True) + b2
    u = (1.0 - h * h) * w2r
    grad = jax.lax.dot_general(u, w1pt_ref[:, :], (((1,), (0,)), ((), ())),
                               preferred_element_type=f32)
    sdf_part = jnp.sum(preds * preds)
    g2 = jnp.sum(grad * grad, axis=1, keepdims=True)
    eik_part = jnp.sum((jnp.sqrt(g2) - 1.0) ** 2)
    gradn_part = jnp.sum((grad - nrows_ref[:, :]) ** 2)

    contrib = ((_SDF_W / _N) * sdf_part
               + (_EIK_W / _N) * eik_part
               + (_GRADN_W / (_N * 3)) * gradn_part
               + w_ori * ori)
    total_ref[:, :] += contrib.reshape(1, 1)


def kernel(surface_points, surface_normals, off_surface_points,
           near_surface_points, W1, b1, W2, b2):
    f32 = jnp.float32
    s = surface_points.astype(f32)
    n = surface_normals.astype(f32)
    s2 = jnp.sum(s * s, axis=1)
    snd = jnp.sum(s * n, axis=1)

    spack = jnp.concatenate(
        [-2.0 * s.T, jnp.ones((1, _N), f32), s2[None, :],
         jnp.zeros((3, _N), f32)], axis=0)
    npack = jnp.concatenate(
        [n, snd[:, None], jnp.zeros((_N, 4), f32)], axis=1)
    srows = jnp.concatenate([s, jnp.zeros((_N, 5), f32)], axis=1)
    nrows = jnp.concatenate([n, jnp.zeros((_N, 5), f32)], axis=1)

    q = jnp.concatenate([off_surface_points, near_surface_points],
                        axis=0).astype(f32)
    q2 = jnp.sum(q * q, axis=1, keepdims=True)
    qrows = jnp.concatenate(
        [q, q2, jnp.ones((2 * _Q, 1), f32), jnp.zeros((2 * _Q, 3), f32)],
        axis=1)

    H = W1.shape[1]
    w1p = jnp.concatenate([W1.astype(f32), jnp.zeros((5, H), f32)], axis=0)
    w1pt = w1p.T
    w2r = W2.astype(f32).reshape(1, H)
    b1r = b1.astype(f32).reshape(1, H)
    b2s = b2.astype(f32).reshape(1, 1)

    out_shape = jax.ShapeDtypeStruct((1, 1), f32)
    scalar_spec = pl.BlockSpec((1, 1), lambda i: (0, 0))
    total = pl.pallas_call(
        _loss_kernel,
        grid=(_GRID,),
        in_specs=[
            pl.BlockSpec((_QB, 8), lambda i: (i, 0)),
            pl.BlockSpec((8, _N), lambda i: (0, 0)),
            pl.BlockSpec((_N, 8), lambda i: (0, 0)),
            pl.BlockSpec((_SB, 8), lambda i: (i, 0)),
            pl.BlockSpec((_SB, 8), lambda i: (i, 0)),
            pl.BlockSpec((8, H), lambda i: (0, 0)),
            pl.BlockSpec((H, 8), lambda i: (0, 0)),
            pl.BlockSpec((1, H), lambda i: (0, 0)),
            pl.BlockSpec((1, H), lambda i: (0, 0)),
            scalar_spec,
        ],
        out_specs=scalar_spec,
        out_shape=out_shape,
    )(qrows, spack, npack, srows, nrows, w1p, w1pt, w2r, b1r, b2s)
    return total[0, 0]


# speedup vs baseline: 18.8655x; 1.3357x over previous
import jax
import jax.numpy as jnp
from jax.experimental import pallas as pl

_SDF_W = 7000.0
_EIK_W = 600.0
_ORI_W = 500.0
_NEAR_ORI_W = 10.0
_GRADN_W = 200.0

_N = 8192
_Q = 8192
_QB = 256
_SB = 128
_GRID = (2 * _Q) // _QB
_BIG = 3e38


def _loss_kernel(q_ref, spack_ref, npack_ref, srows_ref, nrows_ref,
                 w1p_ref, w1pt_ref, w2r_ref, b1_ref, b2_ref, total_ref):
    i = pl.program_id(0)

    @pl.when(i == 0)
    def _init():
        total_ref[:, :] = jnp.zeros((1, 1), jnp.float32)

    f32 = jnp.float32
    qblk = q_ref[:, :]
    spack = spack_ref[:, :]
    npack = npack_ref[:, :]

    def knn_sign(qsub):
        d2 = jax.lax.dot_general(qsub, spack, (((1,), (0,)), ((), ())),
                                 preferred_element_type=f32)
        a1 = d2[:, 0:128]
        a2 = jnp.full(a1.shape, _BIG, f32)
        for c in range(1, _N // 128):
            v = d2[:, 128 * c:128 * (c + 1)]
            hi = jnp.maximum(a1, v)
            a1 = jnp.minimum(a1, v)
            a2 = jnp.minimum(a2, hi)
        m1 = jnp.min(a1, axis=1, keepdims=True)
        m2a = jnp.min(jnp.where(a1 <= m1, _BIG, a1), axis=1, keepdims=True)
        m2b = jnp.min(a2, axis=1, keepdims=True)
        m2 = jnp.minimum(m2a, m2b)
        sel = (d2 <= m2).astype(jnp.bfloat16)
        seln = jax.lax.dot_general(sel, npack, (((1,), (0,)), ((), ())),
                                   preferred_element_type=f32)
        dotsum = (seln[:, 0:1] * qsub[:, 0:1] + seln[:, 1:2] * qsub[:, 1:2]
                  + seln[:, 2:3] * qsub[:, 2:3] - seln[:, 3:4])
        return jnp.sign(dotsum)

    half = _QB // 2
    sgn = jnp.concatenate(
        [knn_sign(qblk[0:half, :]), knn_sign(qblk[half:_QB, :])], axis=0)

    w1p = w1p_ref[:, :]
    w2r = w2r_ref[:, :]
    b1 = b1_ref[:, :]
    b2 = b2_ref[:, :]

    hq = jnp.tanh(jax.lax.dot_general(qblk, w1p, (((1,), (0,)), ((), ())),
                                      preferred_element_type=f32) + b1)
    predq = jnp.sum(hq * w2r, axis=1, keepdims=True) + b2
    ori = jnp.sum(jnp.maximum(-predq * sgn, 0.0))
    w_ori = jnp.where(i < (_Q // _QB), _ORI_W / _Q, _NEAR_ORI_W / _Q)

    sblk = srows_ref[:, :]
    h = jnp.tanh(jax.lax.dot_general(sblk, w1p, (((1,), (0,)), ((), ())),
                                     preferred_element_type=f32) + b1)
    preds = jnp.sum(h * w2r, axis=1, keepdims=True) + b2
    u = (1.0 - h * h) * w2r
    grad = jax.lax.dot_general(u, w1pt_ref[:, :], (((1,), (0,)), ((), ())),
                               preferred_element_type=f32)
    sdf_part = jnp.sum(preds * preds)
    g2 = jnp.sum(grad * grad, axis=1, keepdims=True)
    eik_part = jnp.sum((jnp.sqrt(g2) - 1.0) ** 2)
    gradn_part = jnp.sum((grad - nrows_ref[:, :]) ** 2)

    contrib = ((_SDF_W / _N) * sdf_part
               + (_EIK_W / _N) * eik_part
               + (_GRADN_W / (_N * 3)) * gradn_part
               + w_ori * ori)
    total_ref[:, :] += contrib.reshape(1, 1)


def kernel(surface_points, surface_normals, off_surface_points,
           near_surface_points, W1, b1, W2, b2):
    f32 = jnp.float32
    s = surface_points.astype(f32)
    n = surface_normals.astype(f32)
    s2 = jnp.sum(s * s, axis=1)
    snd = jnp.sum(s * n, axis=1)

    spack = jnp.concatenate(
        [-2.0 * s.T, jnp.ones((1, _N), f32), s2[None, :],
         jnp.zeros((3, _N), f32)], axis=0)
    npack = jnp.concatenate(
        [n, snd[:, None], jnp.zeros((_N, 4), f32)],
        axis=1).astype(jnp.bfloat16)
    srows = jnp.concatenate([s, jnp.zeros((_N, 5), f32)], axis=1)
    nrows = jnp.concatenate([n, jnp.zeros((_N, 5), f32)], axis=1)

    q = jnp.concatenate([off_surface_points, near_surface_points],
                        axis=0).astype(f32)
    q2 = jnp.sum(q * q, axis=1, keepdims=True)
    qrows = jnp.concatenate(
        [q, q2, jnp.ones((2 * _Q, 1), f32), jnp.zeros((2 * _Q, 3), f32)],
        axis=1)

    H = W1.shape[1]
    w1p = jnp.concatenate([W1.astype(f32), jnp.zeros((5, H), f32)], axis=0)
    w1pt = w1p.T
    w2r = W2.astype(f32).reshape(1, H)
    b1r = b1.astype(f32).reshape(1, H)
    b2s = b2.astype(f32).reshape(1, 1)

    out_shape = jax.ShapeDtypeStruct((1, 1), f32)
    scalar_spec = pl.BlockSpec((1, 1), lambda i: (0, 0))
    total = pl.pallas_call(
        _loss_kernel,
        grid=(_GRID,),
        in_specs=[
            pl.BlockSpec((_QB, 8), lambda i: (i, 0)),
            pl.BlockSpec((8, _N), lambda i: (0, 0)),
            pl.BlockSpec((_N, 8), lambda i: (0, 0)),
            pl.BlockSpec((_SB, 8), lambda i: (i, 0)),
            pl.BlockSpec((_SB, 8), lambda i: (i, 0)),
            pl.BlockSpec((8, H), lambda i: (0, 0)),
            pl.BlockSpec((H, 8), lambda i: (0, 0)),
            pl.BlockSpec((1, H), lambda i: (0, 0)),
            pl.BlockSpec((1, H), lambda i: (0, 0)),
            scalar_spec,
        ],
        out_specs=scalar_spec,
        out_shape=out_shape,
    )(qrows, spack, npack, srows, nrows, w1p, w1pt, w2r, b1r, b2s)
    return total[0, 0]


# speedup vs baseline: 19.9982x; 1.0600x over previous
import jax
import jax.numpy as jnp
from jax.experimental import pallas as pl

_SDF_W = 7000.0
_EIK_W = 600.0
_ORI_W = 500.0
_NEAR_ORI_W = 10.0
_GRADN_W = 200.0

_N = 8192
_Q = 8192
_QB = 512
_CH = 128
_SB = 256
_GRID = (2 * _Q) // _QB
_BIG = 3e38


def _loss_kernel(q_ref, spack_ref, npack_ref, srows_ref, nrows_ref,
                 w1p_ref, w1pt_ref, w2r_ref, b1_ref, b2_ref, total_ref):
    i = pl.program_id(0)

    @pl.when(i == 0)
    def _init():
        total_ref[:, :] = jnp.zeros((1, 1), jnp.float32)

    f32 = jnp.float32
    qblk = q_ref[:, :]
    spack = spack_ref[:, :]
    npack = npack_ref[:, :]

    def knn_sign(qsub):
        d2 = jax.lax.dot_general(qsub, spack, (((1,), (0,)), ((), ())),
                                 preferred_element_type=f32)
        a1 = d2[:, 0:128]
        a2 = jnp.full(a1.shape, _BIG, f32)
        for c in range(1, _N // 128):
            v = d2[:, 128 * c:128 * (c + 1)]
            hi = jnp.maximum(a1, v)
            a1 = jnp.minimum(a1, v)
            a2 = jnp.minimum(a2, hi)
        m1 = jnp.min(a1, axis=1, keepdims=True)
        m2a = jnp.min(jnp.where(a1 <= m1, _BIG, a1), axis=1, keepdims=True)
        m2b = jnp.min(a2, axis=1, keepdims=True)
        m2 = jnp.minimum(m2a, m2b)
        sel = (d2 <= m2).astype(jnp.bfloat16)
        seln = jax.lax.dot_general(sel, npack, (((1,), (0,)), ((), ())),
                                   preferred_element_type=f32)
        dotsum = (seln[:, 0:1] * qsub[:, 0:1] + seln[:, 1:2] * qsub[:, 1:2]
                  + seln[:, 2:3] * qsub[:, 2:3] - seln[:, 3:4])
        return jnp.sign(dotsum)

    sgn = jnp.concatenate(
        [knn_sign(qblk[c * _CH:(c + 1) * _CH, :])
         for c in range(_QB // _CH)], axis=0)

    w1p = w1p_ref[:, :]
    w2r = w2r_ref[:, :]
    b1 = b1_ref[:, :]
    b2 = b2_ref[:, :]

    hq = jnp.tanh(jax.lax.dot_general(qblk, w1p, (((1,), (0,)), ((), ())),
                                      preferred_element_type=f32) + b1)
    predq = jnp.sum(hq * w2r, axis=1, keepdims=True) + b2
    ori = jnp.sum(jnp.maximum(-predq * sgn, 0.0))
    w_ori = jnp.where(i < (_Q // _QB), _ORI_W / _Q, _NEAR_ORI_W / _Q)

    sblk = srows_ref[:, :]
    h = jnp.tanh(jax.lax.dot_general(sblk, w1p, (((1,), (0,)), ((), ())),
                                     preferred_element_type=f32) + b1)
    preds = jnp.sum(h * w2r, axis=1, keepdims=True) + b2
    u = (1.0 - h * h) * w2r
    grad = jax.lax.dot_general(u, w1pt_ref[:, :], (((1,), (0,)), ((), ())),
                               preferred_element_type=f32)
    sdf_part = jnp.sum(preds * preds)
    g2 = jnp.sum(grad * grad, axis=1, keepdims=True)
    eik_part = jnp.sum((jnp.sqrt(g2) - 1.0) ** 2)
    gradn_part = jnp.sum((grad - nrows_ref[:, :]) ** 2)

    contrib = ((_SDF_W / _N) * sdf_part
               + (_EIK_W / _N) * eik_part
               + (_GRADN_W / (_N * 3)) * gradn_part
               + w_ori * ori)
    total_ref[:, :] += contrib.reshape(1, 1)


def kernel(surface_points, surface_normals, off_surface_points,
           near_surface_points, W1, b1, W2, b2):
    f32 = jnp.float32
    s = surface_points.astype(f32)
    n = surface_normals.astype(f32)
    s2 = jnp.sum(s * s, axis=1)
    snd = jnp.sum(s * n, axis=1)

    spack = jnp.concatenate(
        [-2.0 * s.T, jnp.ones((1, _N), f32), s2[None, :],
         jnp.zeros((3, _N), f32)], axis=0)
    npack = jnp.concatenate(
        [n, snd[:, None], jnp.zeros((_N, 4), f32)],
        axis=1).astype(jnp.bfloat16)
    srows = jnp.concatenate([s, jnp.zeros((_N, 5), f32)], axis=1)
    nrows = jnp.concatenate([n, jnp.zeros((_N, 5), f32)], axis=1)

    q = jnp.concatenate([off_surface_points, near_surface_points],
                        axis=0).astype(f32)
    q2 = jnp.sum(q * q, axis=1, keepdims=True)
    qrows = jnp.concatenate(
        [q, q2, jnp.ones((2 * _Q, 1), f32), jnp.zeros((2 * _Q, 3), f32)],
        axis=1)

    H = W1.shape[1]
    w1p = jnp.concatenate([W1.astype(f32), jnp.zeros((5, H), f32)], axis=0)
    w1pt = w1p.T
    w2r = W2.astype(f32).reshape(1, H)
    b1r = b1.astype(f32).reshape(1, H)
    b2s = b2.astype(f32).reshape(1, 1)

    out_shape = jax.ShapeDtypeStruct((1, 1), f32)
    scalar_spec = pl.BlockSpec((1, 1), lambda i: (0, 0))
    total = pl.pallas_call(
        _loss_kernel,
        grid=(_GRID,),
        in_specs=[
            pl.BlockSpec((_QB, 8), lambda i: (i, 0)),
            pl.BlockSpec((8, _N), lambda i: (0, 0)),
            pl.BlockSpec((_N, 8), lambda i: (0, 0)),
            pl.BlockSpec((_SB, 8), lambda i: (i, 0)),
            pl.BlockSpec((_SB, 8), lambda i: (i, 0)),
            pl.BlockSpec((8, H), lambda i: (0, 0)),
            pl.BlockSpec((H, 8), lambda i: (0, 0)),
            pl.BlockSpec((1, H), lambda i: (0, 0)),
            pl.BlockSpec((1, H), lambda i: (0, 0)),
            scalar_spec,
        ],
        out_specs=scalar_spec,
        out_shape=out_shape,
    )(qrows, spack, npack, srows, nrows, w1p, w1pt, w2r, b1r, b2s)
    return total[0, 0]


# speedup vs baseline: 20.6414x; 1.0322x over previous
import jax
import jax.numpy as jnp
from jax.experimental import pallas as pl

_SDF_W = 7000.0
_EIK_W = 600.0
_ORI_W = 500.0
_NEAR_ORI_W = 10.0
_GRADN_W = 200.0

_N = 8192
_Q = 8192
_QB = 1024
_CH = 128
_SB = 512
_GRID = (2 * _Q) // _QB
_BIG = 3e38


def _loss_kernel(q_ref, spack_ref, npack_ref, srows_ref, nrows_ref,
                 w1p_ref, w1pt_ref, w2r_ref, b1_ref, b2_ref, total_ref):
    i = pl.program_id(0)

    @pl.when(i == 0)
    def _init():
        total_ref[:, :] = jnp.zeros((1, 1), jnp.float32)

    f32 = jnp.float32
    qblk = q_ref[:, :]
    spack = spack_ref[:, :]
    npack = npack_ref[:, :]

    def knn_sign(qsub):
        d2 = jax.lax.dot_general(qsub, spack, (((1,), (0,)), ((), ())),
                                 preferred_element_type=f32)
        a1 = d2[:, 0:128]
        a2 = jnp.full(a1.shape, _BIG, f32)
        for c in range(1, _N // 128):
            v = d2[:, 128 * c:128 * (c + 1)]
            hi = jnp.maximum(a1, v)
            a1 = jnp.minimum(a1, v)
            a2 = jnp.minimum(a2, hi)
        m1 = jnp.min(a1, axis=1, keepdims=True)
        m2a = jnp.min(jnp.where(a1 <= m1, _BIG, a1), axis=1, keepdims=True)
        m2b = jnp.min(a2, axis=1, keepdims=True)
        m2 = jnp.minimum(m2a, m2b)
        sel = (d2 <= m2).astype(jnp.bfloat16)
        seln = jax.lax.dot_general(sel, npack, (((1,), (0,)), ((), ())),
                                   preferred_element_type=f32)
        dotsum = (seln[:, 0:1] * qsub[:, 0:1] + seln[:, 1:2] * qsub[:, 1:2]
                  + seln[:, 2:3] * qsub[:, 2:3] - seln[:, 3:4])
        return jnp.sign(dotsum)

    sgn = jnp.concatenate(
        [knn_sign(qblk[c * _CH:(c + 1) * _CH, :])
         for c in range(_QB // _CH)], axis=0)

    w1p = w1p_ref[:, :]
    w2r = w2r_ref[:, :]
    b1 = b1_ref[:, :]
    b2 = b2_ref[:, :]

    hq = jnp.tanh(jax.lax.dot_general(qblk, w1p, (((1,), (0,)), ((), ())),
                                      preferred_element_type=f32) + b1)
    predq = jnp.sum(hq * w2r, axis=1, keepdims=True) + b2
    ori = jnp.sum(jnp.maximum(-predq * sgn, 0.0))
    w_ori = jnp.where(i < (_Q // _QB), _ORI_W / _Q, _NEAR_ORI_W / _Q)

    sblk = srows_ref[:, :]
    h = jnp.tanh(jax.lax.dot_general(sblk, w1p, (((1,), (0,)), ((), ())),
                                     preferred_element_type=f32) + b1)
    preds = jnp.sum(h * w2r, axis=1, keepdims=True) + b2
    u = (1.0 - h * h) * w2r
    grad = jax.lax.dot_general(u, w1pt_ref[:, :], (((1,), (0,)), ((), ())),
                               preferred_element_type=f32)
    sdf_part = jnp.sum(preds * preds)
    g2 = jnp.sum(grad * grad, axis=1, keepdims=True)
    eik_part = jnp.sum((jnp.sqrt(g2) - 1.0) ** 2)
    gradn_part = jnp.sum((grad - nrows_ref[:, :]) ** 2)

    contrib = ((_SDF_W / _N) * sdf_part
               + (_EIK_W / _N) * eik_part
               + (_GRADN_W / (_N * 3)) * gradn_part
               + w_ori * ori)
    total_ref[:, :] += contrib.reshape(1, 1)


def kernel(surface_points, surface_normals, off_surface_points,
           near_surface_points, W1, b1, W2, b2):
    f32 = jnp.float32
    s = surface_points.astype(f32)
    n = surface_normals.astype(f32)
    s2 = jnp.sum(s * s, axis=1)
    snd = jnp.sum(s * n, axis=1)

    spack = jnp.concatenate(
        [-2.0 * s.T, jnp.ones((1, _N), f32), s2[None, :],
         jnp.zeros((3, _N), f32)], axis=0)
    npack = jnp.concatenate(
        [n, snd[:, None], jnp.zeros((_N, 4), f32)],
        axis=1).astype(jnp.bfloat16)
    srows = jnp.concatenate([s, jnp.zeros((_N, 5), f32)], axis=1)
    nrows = jnp.concatenate([n, jnp.zeros((_N, 5), f32)], axis=1)

    q = jnp.concatenate([off_surface_points, near_surface_points],
                        axis=0).astype(f32)
    q2 = jnp.sum(q * q, axis=1, keepdims=True)
    qrows = jnp.concatenate(
        [q, q2, jnp.ones((2 * _Q, 1), f32), jnp.zeros((2 * _Q, 3), f32)],
        axis=1)

    H = W1.shape[1]
    w1p = jnp.concatenate([W1.astype(f32), jnp.zeros((5, H), f32)], axis=0)
    w1pt = w1p.T
    w2r = W2.astype(f32).reshape(1, H)
    b1r = b1.astype(f32).reshape(1, H)
    b2s = b2.astype(f32).reshape(1, 1)

    out_shape = jax.ShapeDtypeStruct((1, 1), f32)
    scalar_spec = pl.BlockSpec((1, 1), lambda i: (0, 0))
    total = pl.pallas_call(
        _loss_kernel,
        grid=(_GRID,),
        in_specs=[
            pl.BlockSpec((_QB, 8), lambda i: (i, 0)),
            pl.BlockSpec((8, _N), lambda i: (0, 0)),
            pl.BlockSpec((_N, 8), lambda i: (0, 0)),
            pl.BlockSpec((_SB, 8), lambda i: (i, 0)),
            pl.BlockSpec((_SB, 8), lambda i: (i, 0)),
            pl.BlockSpec((8, H), lambda i: (0, 0)),
            pl.BlockSpec((H, 8), lambda i: (0, 0)),
            pl.BlockSpec((1, H), lambda i: (0, 0)),
            pl.BlockSpec((1, H), lambda i: (0, 0)),
            scalar_spec,
        ],
        out_specs=scalar_spec,
        out_shape=out_shape,
    )(qrows, spack, npack, srows, nrows, w1p, w1pt, w2r, b1r, b2s)
    return total[0, 0]


# speedup vs baseline: 22.2476x; 1.0778x over previous
import jax
import jax.numpy as jnp
from jax.experimental import pallas as pl

_SDF_W = 7000.0
_EIK_W = 600.0
_ORI_W = 500.0
_NEAR_ORI_W = 10.0
_GRADN_W = 200.0

_N = 8192
_Q = 8192
_QB = 1024
_CH = 256
_SB = 512
_GRID = (2 * _Q) // _QB
_BIG = 3e38


def _loss_kernel(q_ref, spack_ref, npack_ref, srows_ref, nrows_ref,
                 w1p_ref, w1pt_ref, w2r_ref, b1_ref, b2_ref, total_ref):
    i = pl.program_id(0)

    @pl.when(i == 0)
    def _init():
        total_ref[:, :] = jnp.zeros((1, 1), jnp.float32)

    f32 = jnp.float32
    qblk = q_ref[:, :]
    spack = spack_ref[:, :]
    npack = npack_ref[:, :]

    def knn_sign(qsub):
        d2 = jax.lax.dot_general(qsub, spack, (((1,), (0,)), ((), ())),
                                 preferred_element_type=f32)
        a1 = d2[:, 0:128]
        a2 = jnp.full(a1.shape, _BIG, f32)
        for c in range(1, _N // 128):
            v = d2[:, 128 * c:128 * (c + 1)]
            hi = jnp.maximum(a1, v)
            a1 = jnp.minimum(a1, v)
            a2 = jnp.minimum(a2, hi)
        m1 = jnp.min(a1, axis=1, keepdims=True)
        m2a = jnp.min(jnp.where(a1 <= m1, _BIG, a1), axis=1, keepdims=True)
        m2b = jnp.min(a2, axis=1, keepdims=True)
        m2 = jnp.minimum(m2a, m2b)
        sel = (d2 <= m2).astype(jnp.bfloat16)
        seln = jax.lax.dot_general(sel, npack, (((1,), (0,)), ((), ())),
                                   preferred_element_type=f32)
        dotsum = (seln[:, 0:1] * qsub[:, 0:1] + seln[:, 1:2] * qsub[:, 1:2]
                  + seln[:, 2:3] * qsub[:, 2:3] - seln[:, 3:4])
        return jnp.sign(dotsum)

    sgn = jnp.concatenate(
        [knn_sign(qblk[c * _CH:(c + 1) * _CH, :])
         for c in range(_QB // _CH)], axis=0)

    w1p = w1p_ref[:, :]
    w2r = w2r_ref[:, :]
    b1 = b1_ref[:, :]
    b2 = b2_ref[:, :]

    hq = jnp.tanh(jax.lax.dot_general(qblk, w1p, (((1,), (0,)), ((), ())),
                                      preferred_element_type=f32) + b1)
    predq = jnp.sum(hq * w2r, axis=1, keepdims=True) + b2
    ori = jnp.sum(jnp.maximum(-predq * sgn, 0.0))
    w_ori = jnp.where(i < (_Q // _QB), _ORI_W / _Q, _NEAR_ORI_W / _Q)

    sblk = srows_ref[:, :]
    h = jnp.tanh(jax.lax.dot_general(sblk, w1p, (((1,), (0,)), ((), ())),
                                     preferred_element_type=f32) + b1)
    preds = jnp.sum(h * w2r, axis=1, keepdims=True) + b2
    u = (1.0 - h * h) * w2r
    grad = jax.lax.dot_general(u, w1pt_ref[:, :], (((1,), (0,)), ((), ())),
                               preferred_element_type=f32)
    sdf_part = jnp.sum(preds * preds)
    g2 = jnp.sum(grad * grad, axis=1, keepdims=True)
    eik_part = jnp.sum((jnp.sqrt(g2) - 1.0) ** 2)
    gradn_part = jnp.sum((grad - nrows_ref[:, :]) ** 2)

    contrib = ((_SDF_W / _N) * sdf_part
               + (_EIK_W / _N) * eik_part
               + (_GRADN_W / (_N * 3)) * gradn_part
               + w_ori * ori)
    total_ref[:, :] += contrib.reshape(1, 1)


def kernel(surface_points, surface_normals, off_surface_points,
           near_surface_points, W1, b1, W2, b2):
    f32 = jnp.float32
    s = surface_points.astype(f32)
    n = surface_normals.astype(f32)
    s2 = jnp.sum(s * s, axis=1)
    snd = jnp.sum(s * n, axis=1)

    spack = jnp.concatenate(
        [-2.0 * s.T, jnp.ones((1, _N), f32), s2[None, :],
         jnp.zeros((3, _N), f32)], axis=0)
    npack = jnp.concatenate(
        [n, snd[:, None], jnp.zeros((_N, 4), f32)],
        axis=1).astype(jnp.bfloat16)
    srows = jnp.concatenate([s, jnp.zeros((_N, 5), f32)], axis=1)
    nrows = jnp.concatenate([n, jnp.zeros((_N, 5), f32)], axis=1)

    q = jnp.concatenate([off_surface_points, near_surface_points],
                        axis=0).astype(f32)
    q2 = jnp.sum(q * q, axis=1, keepdims=True)
    qrows = jnp.concatenate(
        [q, q2, jnp.ones((2 * _Q, 1), f32), jnp.zeros((2 * _Q, 3), f32)],
        axis=1)

    H = W1.shape[1]
    w1p = jnp.concatenate([W1.astype(f32), jnp.zeros((5, H), f32)], axis=0)
    w1pt = w1p.T
    w2r = W2.astype(f32).reshape(1, H)
    b1r = b1.astype(f32).reshape(1, H)
    b2s = b2.astype(f32).reshape(1, 1)

    out_shape = jax.ShapeDtypeStruct((1, 1), f32)
    scalar_spec = pl.BlockSpec((1, 1), lambda i: (0, 0))
    total = pl.pallas_call(
        _loss_kernel,
        grid=(_GRID,),
        in_specs=[
            pl.BlockSpec((_QB, 8), lambda i: (i, 0)),
            pl.BlockSpec((8, _N), lambda i: (0, 0)),
            pl.BlockSpec((_N, 8), lambda i: (0, 0)),
            pl.BlockSpec((_SB, 8), lambda i: (i, 0)),
            pl.BlockSpec((_SB, 8), lambda i: (i, 0)),
            pl.BlockSpec((8, H), lambda i: (0, 0)),
            pl.BlockSpec((H, 8), lambda i: (0, 0)),
            pl.BlockSpec((1, H), lambda i: (0, 0)),
            pl.BlockSpec((1, H), lambda i: (0, 0)),
            scalar_spec,
        ],
        out_specs=scalar_spec,
        out_shape=out_shape,
    )(qrows, spack, npack, srows, nrows, w1p, w1pt, w2r, b1r, b2s)
    return total[0, 0]
